# Initial kernel scaffold; baseline (speedup 1.0000x reference)
import jax
import jax.numpy as jnp
from jax.experimental import pallas as pl


def kernel(card_indices, table):
    raise NotImplementedError("write your pallas kernel here")


# speedup vs baseline: 2.9475x; 2.9475x over previous
import functools

import jax
import jax.numpy as jnp
from jax import lax
from jax.experimental import pallas as pl
from jax.experimental.pallas import tpu as pltpu
from jax.experimental.pallas import tpu_sc as plsc

_EMBED_DIM = 32

_info = plsc.get_sparse_core_info()
_NC, _NS = _info.num_cores, _info.num_subcores
_NW = _NC * _NS

_CHUNK = 1024


def _gather_kernel(n_flat, n_chunks):
    mesh = plsc.VectorSubcoreMesh(core_axis_name="c", subcore_axis_name="s")
    b_per_w = n_flat // _NW

    @functools.partial(
        pl.kernel,
        out_type=jax.ShapeDtypeStruct((n_flat, _EMBED_DIM), jnp.float32),
        mesh=mesh,
        scratch_types=[
            pltpu.VMEM((_CHUNK,), jnp.int32),
            pltpu.VMEM((_CHUNK, _EMBED_DIM), jnp.float32),
            pltpu.SemaphoreType.DMA,
        ],
        compiler_params=pltpu.CompilerParams(use_tc_tiling_on_sc=False),
    )
    def k(idx_hbm, table_hbm, out_hbm, idx_v, rows_v, sem):
        wid = lax.axis_index("s") * _NC + lax.axis_index("c")
        base = wid * b_per_w

        def step(i, carry):
            off = pl.multiple_of(base + i * _CHUNK, _CHUNK)
            pltpu.sync_copy(idx_hbm.at[pl.ds(off, _CHUNK)], idx_v)
            pltpu.async_copy(table_hbm.at[idx_v], rows_v, sem).wait()
            pltpu.sync_copy(rows_v, out_hbm.at[pl.ds(off, _CHUNK)])
            return carry

        lax.fori_loop(0, n_chunks, step, 0)

    return k


def kernel(card_indices, table):
    batch, hist = card_indices.shape
    n_flat = batch * hist
    idx_flat = card_indices.reshape(n_flat).astype(jnp.int32)
    n_chunks = n_flat // (_NW * _CHUNK)
    out = _gather_kernel(n_flat, n_chunks)(idx_flat, table)
    return out.reshape(batch, hist, _EMBED_DIM)


# speedup vs baseline: 3.0068x; 1.0201x over previous
import functools

import jax
import jax.numpy as jnp
from jax import lax
from jax.experimental import pallas as pl
from jax.experimental.pallas import tpu as pltpu
from jax.experimental.pallas import tpu_sc as plsc

_EMBED_DIM = 32

_info = plsc.get_sparse_core_info()
_NC, _NS = _info.num_cores, _info.num_subcores
_NW = _NC * _NS

_CHUNK = 1600
_NBUF = 2


def _gather_kernel(n_flat, n_chunks):
    mesh = plsc.VectorSubcoreMesh(core_axis_name="c", subcore_axis_name="s")
    b_per_w = n_flat // _NW

    @functools.partial(
        pl.kernel,
        out_type=jax.ShapeDtypeStruct((n_flat, _EMBED_DIM), jnp.float32),
        mesh=mesh,
        scratch_types=[
            pltpu.VMEM((_NBUF, _CHUNK), jnp.int32),
            pltpu.VMEM((_NBUF, _CHUNK, _EMBED_DIM), jnp.float32),
            [pltpu.SemaphoreType.DMA] * _NBUF,
            [pltpu.SemaphoreType.DMA] * _NBUF,
            [pltpu.SemaphoreType.DMA] * _NBUF,
        ],
        compiler_params=pltpu.CompilerParams(use_tc_tiling_on_sc=False),
    )
    def k(idx_hbm, table_hbm, out_hbm, idx_v, rows_v, idx_sems, g_sems, w_sems):
        wid = lax.axis_index("s") * _NC + lax.axis_index("c")
        base = wid * b_per_w

        def off(i):
            return pl.multiple_of(base + i * _CHUNK, _CHUNK)

        for i in range(min(_NBUF, n_chunks)):
            pltpu.async_copy(idx_hbm.at[pl.ds(off(i), _CHUNK)], idx_v.at[i],
                             idx_sems[i])

        for i in range(n_chunks):
            b = i % _NBUF
            pltpu.make_async_copy(idx_hbm.at[pl.ds(off(i), _CHUNK)],
                                  idx_v.at[b], idx_sems[b]).wait()
            if i >= _NBUF:
                pltpu.make_async_copy(rows_v.at[b],
                                      out_hbm.at[pl.ds(off(i), _CHUNK)],
                                      w_sems[b]).wait()
            pltpu.async_copy(table_hbm.at[idx_v.at[b]], rows_v.at[b], g_sems[b])
            pltpu.make_async_copy(table_hbm.at[idx_v.at[b]], rows_v.at[b],
                                  g_sems[b]).wait()
            if i + _NBUF < n_chunks:
                pltpu.async_copy(idx_hbm.at[pl.ds(off(i + _NBUF), _CHUNK)],
                                 idx_v.at[b], idx_sems[b])
            pltpu.async_copy(rows_v.at[b], out_hbm.at[pl.ds(off(i), _CHUNK)],
                             w_sems[b])

        for i in range(max(0, n_chunks - _NBUF), n_chunks):
            b = i % _NBUF
            pltpu.make_async_copy(rows_v.at[b],
                                  out_hbm.at[pl.ds(off(i), _CHUNK)],
                                  w_sems[b]).wait()

    return k


def kernel(card_indices, table):
    batch, hist = card_indices.shape
    n_flat = batch * hist
    idx_flat = card_indices.reshape(n_flat).astype(jnp.int32)
    n_chunks = n_flat // (_NW * _CHUNK)
    out = _gather_kernel(n_flat, n_chunks)(idx_flat, table)
    return out.reshape(batch, hist, _EMBED_DIM)
